# baseline (device time: 17388 ns/iter reference)
import jax
import jax.numpy as jnp
from jax import lax
from jax.experimental import pallas as pl
from jax.experimental.pallas import tpu as pltpu

N_DEV = 4
N_EXP = 8
E_PER = 2
CAP = 204.0


def kernel(x, router_W, route_idx, expert_W):
    del router_W
    m, d = x.shape
    _, _, h = expert_W.shape
    chunk = E_PER * d

    x_bf = x.astype(jnp.bfloat16)
    ew_bf = expert_W.astype(jnp.bfloat16)

    def body(x_ref, route_ref, ew_ref, out_ref,
             w_all, hist_all, ws, wr, hs, hr):
        my = lax.axis_index("i")
        left = lax.rem(my + N_DEV - 1, N_DEV)
        right = lax.rem(my + 1, N_DEV)
        diag = lax.rem(my + 2, N_DEV)

        barrier = pltpu.get_barrier_semaphore()
        for nbr in (left, right):
            pl.semaphore_signal(barrier, inc=1, device_id=(nbr,),
                                device_id_type=pl.DeviceIdType.MESH)

        route = route_ref[:, :]
        eids = lax.broadcasted_iota(jnp.int32, (m, N_EXP), 1)
        onehot = (route == eids).astype(jnp.float32)
        hist = jnp.sum(onehot, axis=0, keepdims=True)
        hist_all[pl.ds(my, 1), :] = hist

        pl.semaphore_wait(barrier, 2)

        def rdma(src, dst, ssem, rsem, tgt):
            return pltpu.make_async_remote_copy(
                src_ref=src, dst_ref=dst, send_sem=ssem, recv_sem=rsem,
                device_id=(tgt,), device_id_type=pl.DeviceIdType.MESH)

        def exp_rows(e):
            return w_all.at[pl.ds(e * d, d)]

        h1R = rdma(hist_all.at[pl.ds(my, 1)], hist_all.at[pl.ds(my, 1)],
                   hs.at[0], hr.at[0], right)
        h1L = rdma(hist_all.at[pl.ds(my, 1)], hist_all.at[pl.ds(my, 1)],
                   hs.at[1], hr.at[1], left)
        wA = rdma(ew_ref.at[0], exp_rows(my * E_PER),
                  ws.at[0], wr.at[0], right)
        wC = rdma(ew_ref.at[1], exp_rows(my * E_PER + 1),
                  ws.at[2], wr.at[2], left)
        h1R.start()
        h1L.start()
        wA.start()
        wC.start()

        ri = lax.broadcasted_iota(jnp.int32, (m, m), 0)
        ci = lax.broadcasted_iota(jnp.int32, (m, m), 1)
        tri = (ci < ri).astype(jnp.bfloat16)
        excl = jnp.dot(tri, onehot.astype(jnp.bfloat16),
                       preferred_element_type=jnp.float32)

        h1R.wait_recv()
        h2R = rdma(hist_all.at[pl.ds(left, 1)], hist_all.at[pl.ds(left, 1)],
                   hs.at[2], hr.at[2], right)
        h2R.start()

        wB = rdma(ew_ref.at[1], exp_rows(my * E_PER + 1),
                  ws.at[1], wr.at[1], right)
        wD = rdma(ew_ref.at[0], exp_rows(my * E_PER),
                  ws.at[3], wr.at[3], left)
        wB.start()
        wD.start()

        h1L.wait_recv()
        h2R.wait_recv()

        H = hist_all[:, :]
        lower = (lax.broadcasted_iota(jnp.int32, (N_DEV, N_EXP), 0)
                 < my).astype(jnp.float32)
        offs = jnp.sum(H * lower, axis=0, keepdims=True)
        rank = excl + offs
        mask = onehot * (rank < CAP).astype(jnp.float32)

        xb = x_ref[:, :]

        def masked_x(e):
            m_e = jnp.sum(mask * (eids == e).astype(jnp.float32),
                          axis=1, keepdims=True)
            return xb * m_e.astype(jnp.bfloat16)

        def exp_gemm(e, acc):
            w_e = w_all[pl.ds(e * d, d), :]
            return acc + jnp.dot(masked_x(e), w_e,
                                 preferred_element_type=jnp.float32)

        w2R = rdma(exp_rows(left * E_PER), exp_rows(left * E_PER),
                   ws.at[4], wr.at[4], right)
        w2L = rdma(exp_rows(right * E_PER + 1), exp_rows(right * E_PER + 1),
                   ws.at[5], wr.at[5], left)
        wA.wait_recv()
        w2R.start()
        wC.wait_recv()
        w2L.start()

        acc = jnp.zeros((m, h), jnp.float32)
        acc = acc + jnp.dot(masked_x(my * E_PER), ew_ref[0, :, :],
                            preferred_element_type=jnp.float32)
        acc = acc + jnp.dot(masked_x(my * E_PER + 1), ew_ref[1, :, :],
                            preferred_element_type=jnp.float32)
        acc = exp_gemm(left * E_PER, acc)
        acc = exp_gemm(right * E_PER + 1, acc)
        wB.wait_recv()
        acc = exp_gemm(left * E_PER + 1, acc)
        wD.wait_recv()
        acc = exp_gemm(right * E_PER, acc)
        w2R.wait_recv()
        acc = exp_gemm(diag * E_PER, acc)
        w2L.wait_recv()
        acc = exp_gemm(diag * E_PER + 1, acc)

        out_ref[:, :] = acc

        for r in (h1R, h1L, h2R, wA, wB, wC, wD, w2R, w2L):
            r.wait_send()

    return pl.pallas_call(
        body,
        out_shape=jax.ShapeDtypeStruct((m, h), jnp.float32),
        in_specs=[
            pl.BlockSpec(memory_space=pltpu.VMEM),
            pl.BlockSpec(memory_space=pltpu.VMEM),
            pl.BlockSpec(memory_space=pltpu.VMEM),
        ],
        out_specs=pl.BlockSpec(memory_space=pltpu.VMEM),
        scratch_shapes=[
            pltpu.VMEM((N_EXP * d, h), jnp.bfloat16),
            pltpu.VMEM((N_DEV, N_EXP), jnp.float32),
            pltpu.SemaphoreType.DMA((6,)),
            pltpu.SemaphoreType.DMA((6,)),
            pltpu.SemaphoreType.DMA((3,)),
            pltpu.SemaphoreType.DMA((3,)),
        ],
        compiler_params=pltpu.CompilerParams(collective_id=0),
    )(x_bf, route_idx, ew_bf)
